# baseline (speedup 1.0000x reference)
import jax
import jax.numpy as jnp
from jax import lax
from jax.experimental import pallas as pl
from jax.experimental.pallas import tpu as pltpu
from jax.experimental.pallas import tpu_sc as plsc

NUM_NODES = 100000
EMBED_DIM = 64
SUB = 128
CHUNK = 512
NSUB = CHUNK // SUB
NFULL = NUM_NODES // CHUNK
TAIL = NUM_NODES - NFULL * CHUNK
TAIL_SUBS = (128, 32)
NUM_WORKERS = 32
KSTEPS = -(-NFULL // NUM_WORKERS)
TAIL_WID = 3

_mesh = plsc.VectorSubcoreMesh(core_axis_name="c", subcore_axis_name="s")


@pl.kernel(
    mesh=_mesh,
    compiler_params=pltpu.CompilerParams(use_tc_tiling_on_sc=False),
    out_type=jax.ShapeDtypeStruct((NUM_NODES, EMBED_DIM), jnp.float32),
    scratch_types=[
        pltpu.VMEM((CHUNK,), jnp.int32),
        pltpu.VMEM((CHUNK,), jnp.int32),
        pltpu.VMEM((CHUNK, EMBED_DIM), jnp.float32),
        pltpu.VMEM((CHUNK, EMBED_DIM), jnp.float32),
        pltpu.SemaphoreType.DMA,
        pltpu.SemaphoreType.DMA,
    ],
)
def _gather_kernel(idx_hbm, table_hbm, out_hbm, idx0, idx1, rows0, rows1,
                   sem0, sem1):
    wid = lax.axis_index("s") * 2 + lax.axis_index("c")
    idx_b = (idx0, idx1)
    rows_b = (rows0, rows1)
    sem_b = (sem0, sem1)

    def start(k):
        chunk = k * NUM_WORKERS + wid

        @pl.when(chunk < NFULL)
        def _():
            b = k % 2
            base = chunk * CHUNK
            pltpu.sync_copy(idx_hbm.at[pl.ds(base, CHUNK)], idx_b[b])
            for j in range(NSUB):
                pltpu.async_copy(
                    table_hbm.at[idx_b[b].at[pl.ds(j * SUB, SUB)]],
                    rows_b[b].at[pl.ds(j * SUB, SUB)],
                    sem_b[b],
                )

    def finish(k):
        chunk = k * NUM_WORKERS + wid

        @pl.when(chunk < NFULL)
        def _():
            b = k % 2
            base = chunk * CHUNK
            for j in range(NSUB):
                pltpu.make_async_copy(
                    table_hbm.at[idx_b[b].at[pl.ds(j * SUB, SUB)]],
                    rows_b[b].at[pl.ds(j * SUB, SUB)],
                    sem_b[b],
                ).wait()
            pltpu.sync_copy(rows_b[b], out_hbm.at[pl.ds(base, CHUNK)])

    start(0)
    for k in range(KSTEPS):
        if k + 1 < KSTEPS:
            start(k + 1)
        finish(k)

    @pl.when(wid == TAIL_WID)
    def _():
        base = NFULL * CHUNK
        off = 0
        for n in TAIL_SUBS:
            pltpu.sync_copy(idx_hbm.at[pl.ds(base + off, n)],
                            idx0.at[pl.ds(off, n)])
            pltpu.async_copy(
                table_hbm.at[idx0.at[pl.ds(off, n)]],
                rows0.at[pl.ds(off, n)],
                sem0,
            ).wait()
            off += n
        pltpu.sync_copy(rows0.at[pl.ds(0, TAIL)],
                        out_hbm.at[pl.ds(base, TAIL)])


def kernel(type_indices, type_embedding):
    return _gather_kernel(type_indices.astype(jnp.int32), type_embedding)


# speedup vs baseline: 2.1760x; 2.1760x over previous
import jax
import jax.numpy as jnp
from jax import lax
from jax.experimental import pallas as pl
from jax.experimental.pallas import tpu as pltpu
from jax.experimental.pallas import tpu_sc as plsc

NUM_NODES = 100000
EMBED_DIM = 64
SUB = 128
CHUNK = 512
NSUB = CHUNK // SUB
NFULL = NUM_NODES // CHUNK
TAIL = NUM_NODES - NFULL * CHUNK
TAIL_SUBS = (128, 32)
NUM_WORKERS = 32
KSTEPS = -(-NFULL // NUM_WORKERS)
TAIL_WID = 3

_mesh = plsc.VectorSubcoreMesh(core_axis_name="c", subcore_axis_name="s")


@pl.kernel(
    mesh=_mesh,
    compiler_params=pltpu.CompilerParams(use_tc_tiling_on_sc=False),
    out_type=jax.ShapeDtypeStruct((NUM_NODES, EMBED_DIM), jnp.float32),
    scratch_types=[
        pltpu.VMEM((CHUNK,), jnp.int32),
        pltpu.VMEM((CHUNK,), jnp.int32),
        pltpu.VMEM((CHUNK, EMBED_DIM), jnp.float32),
        pltpu.VMEM((CHUNK, EMBED_DIM), jnp.float32),
        pltpu.VMEM_SHARED((64, EMBED_DIM), jnp.float32),
        pltpu.SemaphoreType.DMA,
        pltpu.SemaphoreType.DMA,
    ],
)
def _gather_kernel(idx_hbm, table_hbm, out_hbm, idx0, idx1, rows0, rows1,
                   tab_sh, sem0, sem1):
    wid = lax.axis_index("s") * 2 + lax.axis_index("c")

    @pl.when(lax.axis_index("s") == 0)
    def _():
        pltpu.sync_copy(table_hbm, tab_sh)

    plsc.subcore_barrier()
    idx_b = (idx0, idx1)
    rows_b = (rows0, rows1)
    sem_b = (sem0, sem1)

    def start(k):
        chunk = k * NUM_WORKERS + wid

        @pl.when(chunk < NFULL)
        def _():
            b = k % 2
            base = chunk * CHUNK
            pltpu.sync_copy(idx_hbm.at[pl.ds(base, CHUNK)], idx_b[b])
            for j in range(NSUB):
                pltpu.async_copy(
                    tab_sh.at[idx_b[b].at[pl.ds(j * SUB, SUB)]],
                    rows_b[b].at[pl.ds(j * SUB, SUB)],
                    sem_b[b],
                )

    def finish(k):
        chunk = k * NUM_WORKERS + wid

        @pl.when(chunk < NFULL)
        def _():
            b = k % 2
            base = chunk * CHUNK
            for j in range(NSUB):
                pltpu.make_async_copy(
                    tab_sh.at[idx_b[b].at[pl.ds(j * SUB, SUB)]],
                    rows_b[b].at[pl.ds(j * SUB, SUB)],
                    sem_b[b],
                ).wait()
            pltpu.sync_copy(rows_b[b], out_hbm.at[pl.ds(base, CHUNK)])

    start(0)
    for k in range(KSTEPS):
        if k + 1 < KSTEPS:
            start(k + 1)
        finish(k)

    @pl.when(wid == TAIL_WID)
    def _():
        base = NFULL * CHUNK
        off = 0
        for n in TAIL_SUBS:
            pltpu.sync_copy(idx_hbm.at[pl.ds(base + off, n)],
                            idx0.at[pl.ds(off, n)])
            pltpu.async_copy(
                tab_sh.at[idx0.at[pl.ds(off, n)]],
                rows0.at[pl.ds(off, n)],
                sem0,
            ).wait()
            off += n
        pltpu.sync_copy(rows0.at[pl.ds(0, TAIL)],
                        out_hbm.at[pl.ds(base, TAIL)])


def kernel(type_indices, type_embedding):
    return _gather_kernel(type_indices.astype(jnp.int32), type_embedding)
